# baseline (speedup 1.0000x reference)
import jax
import jax.numpy as jnp
from jax import lax
from jax.experimental import pallas as pl
from jax.experimental.pallas import tpu as pltpu, tpu_sc as plsc

_B = 16384
_D = 64
_NC = 2
_NS = 16
_NW = _NC * _NS
_BPW = _B // _NW
_CHUNK = 128
_NCHUNK = _BPW // _CHUNK
_NGATHER = 6
_NTASK = _NGATHER * _NCHUNK
_NBUF = 4

_mesh = plsc.VectorSubcoreMesh(
    core_axis_name="c", subcore_axis_name="s",
    num_cores=_NC, num_subcores=_NS,
)


def _body(user_hbm, track_hbm, xu, xtp, xtn, xup, xun, xta,
          u_out, tp_out, tn_out, up_out, un_out, ta_out,
          i0, i1, i2, i3, i4, i5, rows_v, sem):
    wid = lax.axis_index("s") * _NC + lax.axis_index("c")
    base = wid * _BPW

    tables = (user_hbm, track_hbm, track_hbm, user_hbm, user_hbm, track_hbm)
    idx_in = (xu, xtp, xtn, xup, xun, xta)
    idx_v = (i0, i1, i2, i3, i4, i5)
    outs = (u_out, tp_out, tn_out, up_out, un_out, ta_out)

    for g in range(_NGATHER):
        pltpu.sync_copy(idx_in[g].at[wid], idx_v[g])

    def task(t):
        g, c = divmod(t, _NCHUNK)
        return tables[g], idx_v[g].at[c], outs[g], base + c * _CHUNK

    descs = {}
    for t in range(_NBUF):
        tab, idx, _, _ = task(t)
        descs[t] = pltpu.async_copy(tab.at[idx], rows_v.at[t % _NBUF], sem)
    for t in range(_NTASK):
        descs.pop(t).wait()
        _, _, out, off = task(t)
        pltpu.sync_copy(rows_v.at[t % _NBUF], out.at[pl.ds(off, _CHUNK)])
        nt = t + _NBUF
        if nt < _NTASK:
            tab, idx, _, _ = task(nt)
            descs[nt] = pltpu.async_copy(tab.at[idx], rows_v.at[nt % _NBUF], sem)


_out_struct = jax.ShapeDtypeStruct((_B, _D), jnp.float32)

_gather6 = pl.kernel(
    _body,
    out_type=(_out_struct,) * _NGATHER,
    mesh=_mesh,
    scratch_types=(
        [pltpu.VMEM((_NCHUNK, _CHUNK), jnp.int32)] * _NGATHER
        + [pltpu.VMEM((_NBUF, _CHUNK, _D), jnp.float32),
           pltpu.SemaphoreType.DMA]
    ),
    compiler_params=pltpu.CompilerParams(use_tc_tiling_on_sc=False),
)


def kernel(user_mat, track_mat, x_user, x_track_pos, x_track_neg,
           x_user_pos, x_user_neg, x_track_anchor):
    def prep(x):
        return jnp.reshape(x.astype(jnp.int32), (_NW, _NCHUNK, _CHUNK))

    return _gather6(
        user_mat, track_mat,
        prep(x_user), prep(x_track_pos), prep(x_track_neg),
        prep(x_user_pos), prep(x_user_neg), prep(x_track_anchor),
    )
